# speedup vs baseline: 3.9513x; 3.9513x over previous
import functools

import jax
import jax.numpy as jnp
from jax import lax
from jax.experimental import pallas as pl
from jax.experimental.pallas import tpu as pltpu
from jax.experimental.pallas import tpu_sc as plsc

VOCAB = 100000
MAXLEN = 200
EMBED = 128
BATCH = 1024

NC = 2
NS = 16
L = 16
NW = NC * NS
SEQ_PER_W = BATCH // NW
HALF = MAXLEN // 2


@jax.jit
def _embed(x3, token_table, pos_table):
    mesh = plsc.VectorSubcoreMesh(
        core_axis_name="c", subcore_axis_name="s", num_cores=NC, num_subcores=NS
    )

    @functools.partial(
        pl.kernel,
        mesh=mesh,
        out_type=jax.ShapeDtypeStruct((BATCH * MAXLEN, EMBED), jnp.float32),
        scratch_types=[
            pltpu.VMEM((2, HALF), jnp.int32),
            pltpu.VMEM((MAXLEN, EMBED), jnp.float32),
            pltpu.VMEM((MAXLEN, EMBED), jnp.float32),
            pltpu.SemaphoreType.DMA,
        ],
    )
    def k(x_hbm, table_hbm, pos_hbm, out_hbm, idx_v, rows_v, pos_vm, sem):
        wid = lax.axis_index("s") * NC + lax.axis_index("c")
        pltpu.sync_copy(pos_hbm, pos_vm)

        def seq_body(t, carry):
            seq = wid * SEQ_PER_W + t
            pltpu.sync_copy(x_hbm.at[seq], idx_v)
            cp0 = pltpu.async_copy(
                table_hbm.at[idx_v.at[0]], rows_v.at[pl.ds(0, HALF)], sem
            )
            cp1 = pltpu.async_copy(
                table_hbm.at[idx_v.at[1]], rows_v.at[pl.ds(HALF, HALF)], sem
            )
            cp0.wait()
            cp1.wait()

            def add_body(r, c2):
                for c in range(EMBED // L):
                    sl = pl.ds(c * L, L)
                    rows_v[r, sl] = rows_v[r, sl] + pos_vm[r, sl]
                return c2

            lax.fori_loop(0, MAXLEN, add_body, 0)
            pltpu.sync_copy(rows_v, out_hbm.at[pl.ds(seq * MAXLEN, MAXLEN)])
            return carry

        lax.fori_loop(0, SEQ_PER_W, seq_body, 0)

    return k(x3, token_table, pos_table)


def kernel(x, token_table, pos_table):
    x3 = x.astype(jnp.int32).reshape(BATCH, 2, HALF)
    out = _embed(x3, token_table, pos_table)
    return out.reshape(BATCH, MAXLEN, EMBED)


# speedup vs baseline: 4.2410x; 1.0733x over previous
import functools

import jax
import jax.numpy as jnp
from jax import lax
from jax.experimental import pallas as pl
from jax.experimental.pallas import tpu as pltpu
from jax.experimental.pallas import tpu_sc as plsc

VOCAB = 100000
MAXLEN = 200
EMBED = 128
BATCH = 1024

NC = 2
NS = 16
L = 16
NW = NC * NS
C = 256
CHUNKS = BATCH // C
UNITS = MAXLEN * CHUNKS // NW
GI = 128


@jax.jit
def _embed(x4, token_table, pos_table):
    mesh = plsc.VectorSubcoreMesh(
        core_axis_name="c", subcore_axis_name="s", num_cores=NC, num_subcores=NS
    )

    @functools.partial(
        pl.kernel,
        mesh=mesh,
        out_type=jax.ShapeDtypeStruct((BATCH, MAXLEN * EMBED), jnp.float32),
        scratch_types=[
            pltpu.VMEM((C // GI, GI), jnp.int32),
            pltpu.VMEM((C // GI, GI), jnp.int32),
            pltpu.VMEM((C, EMBED), jnp.float32),
            pltpu.VMEM((C, EMBED), jnp.float32),
            pltpu.VMEM((MAXLEN, EMBED), jnp.float32),
            pltpu.SemaphoreType.DMA,
            pltpu.SemaphoreType.DMA,
            pltpu.SemaphoreType.DMA,
            pltpu.SemaphoreType.DMA,
            pltpu.SemaphoreType.DMA,
            pltpu.SemaphoreType.DMA,
        ],
    )
    def k(x_hbm, table_hbm, pos_hbm, out_hbm,
          idx0, idx1, rows0, rows1, pos_vm,
          gsem0, gsem1, wsem0, wsem1, isem0, isem1):
        idx = [idx0, idx1]
        rows = [rows0, rows1]
        gsem = [gsem0, gsem1]
        wsem = [wsem0, wsem1]
        isem = [isem0, isem1]

        wid = lax.axis_index("s") * NC + lax.axis_index("c")
        base_u = wid * UNITS
        pltpu.sync_copy(pos_hbm, pos_vm)

        def unit_pc(u):
            gu = base_u + u
            return gu // CHUNKS, gu % CHUNKS

        def fetch_idx(u, b):
            p, ch = unit_pc(u)
            return pltpu.async_copy(x_hbm.at[p, ch], idx[b], isem[b])

        def start_gather(b):
            return [
                pltpu.async_copy(
                    table_hbm.at[idx[b].at[j]],
                    rows[b].at[pl.ds(j * GI, GI)],
                    gsem[b],
                )
                for j in range(C // GI)
            ]

        fetch_idx(0, 0).wait()
        pending_g = [start_gather(0), None]
        pending_i = [None, fetch_idx(1, 1) if UNITS > 1 else None]
        pending_w = [None, None]

        for u in range(UNITS):
            b = u % 2
            o = 1 - b
            if u + 1 < UNITS:
                pending_i[o].wait()
                if pending_w[o] is not None:
                    pending_w[o].wait()
                    pending_w[o] = None
                pending_g[o] = start_gather(o)
            for cp in pending_g[b]:
                cp.wait()
            if u + 2 < UNITS:
                pending_i[b] = fetch_idx(u + 2, b)

            p, ch = unit_pc(u)
            rv = rows[b]
            pvals = [pos_vm[p, pl.ds(c * L, L)] for c in range(EMBED // L)]

            def add_body(r, carry, rv=rv, pvals=pvals):
                for c in range(EMBED // L):
                    sl = pl.ds(c * L, L)
                    rv[r, sl] = rv[r, sl] + pvals[c]
                return carry

            lax.fori_loop(0, C, add_body, 0)
            pending_w[b] = pltpu.async_copy(
                rv, out_hbm.at[pl.ds(ch * C, C), pl.ds(p * EMBED, EMBED)], wsem[b]
            )

        for b in range(2):
            if pending_w[b] is not None:
                pending_w[b].wait()

    return k(x4, token_table, pos_table)


def kernel(x, token_table, pos_table):
    x4 = x.astype(jnp.int32).T.reshape(MAXLEN, CHUNKS, C // GI, GI)
    out = _embed(x4, token_table, pos_table)
    return out.reshape(BATCH, MAXLEN, EMBED)


# speedup vs baseline: 6.6146x; 1.5597x over previous
import functools

import jax
import jax.numpy as jnp
from jax import lax
from jax.experimental import pallas as pl
from jax.experimental.pallas import tpu as pltpu
from jax.experimental.pallas import tpu_sc as plsc

VOCAB = 100000
MAXLEN = 200
EMBED = 128
BATCH = 1024

NC = 2
NS = 16
L = 16
NW = NC * NS
C = 128
CHUNKS = BATCH // C
PRANGES = NW // CHUNKS
PPW = MAXLEN // PRANGES


@jax.jit
def _embed(x, token_table, pos_table):
    mesh = plsc.VectorSubcoreMesh(
        core_axis_name="c", subcore_axis_name="s", num_cores=NC, num_subcores=NS
    )

    @functools.partial(
        pl.kernel,
        mesh=mesh,
        compiler_params=pltpu.CompilerParams(
            use_tc_tiling_on_sc=False, needs_layout_passes=False
        ),
        out_type=jax.ShapeDtypeStruct((BATCH, MAXLEN * EMBED), jnp.float32),
        scratch_types=[
            pltpu.VMEM((C * MAXLEN,), jnp.int32),
            pltpu.VMEM((C,), jnp.int32),
            pltpu.VMEM((C,), jnp.int32),
            pltpu.VMEM((C, EMBED), jnp.float32),
            pltpu.VMEM((C, EMBED), jnp.float32),
            pltpu.VMEM((MAXLEN, EMBED), jnp.float32),
            pltpu.SemaphoreType.DMA,
            pltpu.SemaphoreType.DMA,
            pltpu.SemaphoreType.DMA,
            pltpu.SemaphoreType.DMA,
            pltpu.SemaphoreType.DMA,
            pltpu.SemaphoreType.DMA,
        ],
    )
    def k(x_hbm, table_hbm, pos_hbm, out_hbm,
          xblk, idx0, idx1, rows0, rows1, pos_vm,
          gsem0, gsem1, wsem0, wsem1, isem0, isem1):
        idx = [idx0, idx1]
        rows = [rows0, rows1]
        gsem = [gsem0, gsem1]
        wsem = [wsem0, wsem1]
        isem = [isem0, isem1]

        wid = lax.axis_index("s") * NC + lax.axis_index("c")
        ch = wid % CHUNKS
        q = wid // CHUNKS
        b0 = ch * C
        p0 = q * PPW

        pltpu.sync_copy(pos_hbm, pos_vm)
        pltpu.sync_copy(x_hbm.at[pl.ds(b0 * MAXLEN, C * MAXLEN)], xblk)

        iota = lax.iota(jnp.int32, L)
        row_base = iota * MAXLEN

        def build_idx(u, b):
            col = row_base + (p0 + u)

            def body(j, carry):
                vals = plsc.load_gather(xblk, [col + j * (L * MAXLEN)])
                idx[b][pl.ds(j * L, L)] = vals
                return carry

            lax.fori_loop(0, C // L, body, 0)

        def start_gather(b):
            return pltpu.async_copy(table_hbm.at[idx[b]], rows[b], gsem[b])

        build_idx(0, 0)
        pending_g = [start_gather(0), None]
        pending_w = [None, None]

        for u in range(PPW):
            b = u % 2
            o = 1 - b
            if u + 1 < PPW:
                build_idx(u + 1, o)
                if pending_w[o] is not None:
                    pending_w[o].wait()
                    pending_w[o] = None
                pending_g[o] = start_gather(o)
            pending_g[b].wait()

            rv = rows[b]
            pvals = [pos_vm[p0 + u, pl.ds(c * L, L)] for c in range(EMBED // L)]

            def add_body(r, carry, rv=rv, pvals=pvals):
                for c in range(EMBED // L):
                    sl = pl.ds(c * L, L)
                    rv[r, sl] = rv[r, sl] + pvals[c]
                return carry

            lax.fori_loop(0, C, add_body, 0)
            pending_w[b] = pltpu.async_copy(
                rv,
                out_hbm.at[pl.ds(b0, C), pl.ds((p0 + u) * EMBED, EMBED)],
                wsem[b],
            )

        for b in range(2):
            if pending_w[b] is not None:
                pending_w[b].wait()

    return k(x, token_table, pos_table)


def kernel(x, token_table, pos_table):
    out = _embed(x.astype(jnp.int32).reshape(-1), token_table, pos_table)
    return out.reshape(BATCH, MAXLEN, EMBED)


# speedup vs baseline: 6.9747x; 1.0544x over previous
import functools

import jax
import jax.numpy as jnp
from jax import lax
from jax.experimental import pallas as pl
from jax.experimental.pallas import tpu as pltpu
from jax.experimental.pallas import tpu_sc as plsc

VOCAB = 100000
MAXLEN = 200
EMBED = 128
BATCH = 1024

NC = 2
NS = 16
L = 16
NW = NC * NS
C = 128
CHUNKS = BATCH // C
PRANGES = NW // CHUNKS
PPW = MAXLEN // PRANGES
NBUF = 3


@jax.jit
def _embed(x_flat, token_table, pos_table):
    mesh = plsc.VectorSubcoreMesh(
        core_axis_name="c", subcore_axis_name="s", num_cores=NC, num_subcores=NS
    )

    @functools.partial(
        pl.kernel,
        mesh=mesh,
        compiler_params=pltpu.CompilerParams(
            use_tc_tiling_on_sc=False, needs_layout_passes=False
        ),
        out_type=jax.ShapeDtypeStruct((BATCH, MAXLEN * EMBED), jnp.float32),
        scratch_types=[
            pltpu.VMEM((C * MAXLEN,), jnp.int32),
            pltpu.VMEM((NBUF, C), jnp.int32),
            pltpu.VMEM((C, EMBED), jnp.float32),
            pltpu.VMEM((C, EMBED), jnp.float32),
            pltpu.VMEM((C, EMBED), jnp.float32),
            pltpu.VMEM((MAXLEN, EMBED), jnp.float32),
            pltpu.SemaphoreType.DMA,
            pltpu.SemaphoreType.DMA,
            pltpu.SemaphoreType.DMA,
            pltpu.SemaphoreType.DMA,
            pltpu.SemaphoreType.DMA,
            pltpu.SemaphoreType.DMA,
        ],
    )
    def k(x_hbm, table_hbm, pos_hbm, out_hbm,
          xblk, idx, rows0, rows1, rows2, pos_vm,
          gsem0, gsem1, gsem2, wsem0, wsem1, wsem2):
        rows = [rows0, rows1, rows2]
        gsem = [gsem0, gsem1, gsem2]
        wsem = [wsem0, wsem1, wsem2]

        wid = lax.axis_index("s") * NC + lax.axis_index("c")
        ch = wid % CHUNKS
        q = wid // CHUNKS
        b0 = ch * C
        p0 = q * PPW

        pltpu.sync_copy(pos_hbm, pos_vm)
        pltpu.sync_copy(x_hbm.at[pl.ds(b0 * MAXLEN, C * MAXLEN)], xblk)

        iota = lax.iota(jnp.int32, L)
        row_base = iota * MAXLEN

        def build_idx(u, b):
            col = row_base + (p0 + u)

            def body(j, carry):
                vals = plsc.load_gather(xblk, [col + j * (L * MAXLEN)])
                idx[b, pl.ds(j * L, L)] = vals
                return carry

            lax.fori_loop(0, C // L, body, 0)

        def start_gather(b):
            return pltpu.async_copy(table_hbm.at[idx.at[b]], rows[b], gsem[b])

        build_idx(0, 0)
        pending_g = [start_gather(0), None, None]
        build_idx(1, 1)
        pending_g[1] = start_gather(1)
        pending_w = [None, None, None]

        for u in range(PPW):
            b = u % NBUF
            if u + 2 < PPW:
                nb = (u + 2) % NBUF
                build_idx(u + 2, nb)
                if pending_w[nb] is not None:
                    pending_w[nb].wait()
                    pending_w[nb] = None
                pending_g[nb] = start_gather(nb)
            pending_g[b].wait()

            rv = rows[b]
            pvals = [pos_vm[p0 + u, pl.ds(c * L, L)] for c in range(EMBED // L)]

            @plsc.parallel_loop(0, C, step=1, unroll=2)
            def add_body(r, rv=rv, pvals=pvals):
                for c in range(EMBED // L):
                    sl = pl.ds(c * L, L)
                    rv[r, sl] = rv[r, sl] + pvals[c]

            pending_w[b] = pltpu.async_copy(
                rv,
                out_hbm.at[pl.ds(b0, C), pl.ds((p0 + u) * EMBED, EMBED)],
                wsem[b],
            )

        for b in range(NBUF):
            if pending_w[b] is not None:
                pending_w[b].wait()

    return k(x_flat, token_table, pos_table)


def kernel(x, token_table, pos_table):
    out = _embed(x.astype(jnp.int32).reshape(-1), token_table, pos_table)
    return out.reshape(BATCH, MAXLEN, EMBED)


# speedup vs baseline: 7.1704x; 1.0281x over previous
import functools

import jax
import jax.numpy as jnp
from jax import lax
from jax.experimental import pallas as pl
from jax.experimental.pallas import tpu as pltpu
from jax.experimental.pallas import tpu_sc as plsc

VOCAB = 100000
MAXLEN = 200
EMBED = 128
BATCH = 1024

NC = 2
NS = 16
L = 16
NW = NC * NS
C = 128
CHUNKS = BATCH // C
PRANGES = NW // CHUNKS
PPW = MAXLEN // PRANGES
NBUF = 3


@jax.jit
def _embed(x_flat, token_table, pos_table):
    mesh = plsc.VectorSubcoreMesh(
        core_axis_name="c", subcore_axis_name="s", num_cores=NC, num_subcores=NS
    )

    @functools.partial(
        pl.kernel,
        mesh=mesh,
        compiler_params=pltpu.CompilerParams(
            use_tc_tiling_on_sc=False, needs_layout_passes=False
        ),
        out_type=jax.ShapeDtypeStruct((BATCH, MAXLEN * EMBED), jnp.float32),
        scratch_types=[
            pltpu.VMEM((C * MAXLEN,), jnp.int32),
            pltpu.VMEM((NBUF, C), jnp.int32),
            pltpu.VMEM((C, EMBED), jnp.float32),
            pltpu.VMEM((C, EMBED), jnp.float32),
            pltpu.VMEM((C, EMBED), jnp.float32),
            pltpu.VMEM((PPW, EMBED), jnp.float32),
            pltpu.SemaphoreType.DMA,
            pltpu.SemaphoreType.DMA,
            pltpu.SemaphoreType.DMA,
            pltpu.SemaphoreType.DMA,
            pltpu.SemaphoreType.DMA,
            pltpu.SemaphoreType.DMA,
        ],
    )
    def k(x_hbm, table_hbm, pos_hbm, out_hbm,
          xblk, idx, rows0, rows1, rows2, pos_vm,
          gsem0, gsem1, gsem2, wsem0, wsem1, wsem2):
        rows = [rows0, rows1, rows2]
        gsem = [gsem0, gsem1, gsem2]
        wsem = [wsem0, wsem1, wsem2]

        wid = lax.axis_index("s") * NC + lax.axis_index("c")
        ch = wid % CHUNKS
        q = wid // CHUNKS
        b0 = ch * C
        p0 = q * PPW

        pltpu.sync_copy(x_hbm.at[pl.ds(b0 * MAXLEN, C * MAXLEN)], xblk)

        iota = lax.iota(jnp.int32, L)
        row_base = iota * MAXLEN

        def build_idx(u, b):
            col = row_base + (p0 + u)

            def body(j, carry):
                vals = plsc.load_gather(xblk, [col + j * (L * MAXLEN)])
                idx[b, pl.ds(j * L, L)] = vals
                return carry

            lax.fori_loop(0, C // L, body, 0)

        def start_gather(b):
            return pltpu.async_copy(table_hbm.at[idx.at[b]], rows[b], gsem[b])

        build_idx(0, 0)
        pending_g = [start_gather(0), None, None]
        build_idx(1, 1)
        pending_g[1] = start_gather(1)
        pending_w = [None, None, None]
        pltpu.sync_copy(pos_hbm.at[pl.ds(p0, PPW)], pos_vm)

        for u in range(PPW):
            b = u % NBUF
            if u + 2 < PPW:
                nb = (u + 2) % NBUF
                build_idx(u + 2, nb)
                if pending_w[nb] is not None:
                    pending_w[nb].wait()
                    pending_w[nb] = None
                pending_g[nb] = start_gather(nb)
            pending_g[b].wait()

            rv = rows[b]
            pvals = [pos_vm[u, pl.ds(c * L, L)] for c in range(EMBED // L)]

            @plsc.parallel_loop(0, C, step=1, unroll=4)
            def add_body(r, rv=rv, pvals=pvals):
                for c in range(EMBED // L):
                    sl = pl.ds(c * L, L)
                    rv[r, sl] = rv[r, sl] + pvals[c]

            pending_w[b] = pltpu.async_copy(
                rv,
                out_hbm.at[pl.ds(b0, C), pl.ds((p0 + u) * EMBED, EMBED)],
                wsem[b],
            )

        for b in range(NBUF):
            if pending_w[b] is not None:
                pending_w[b].wait()

    return k(x_flat, token_table, pos_table)


def kernel(x, token_table, pos_table):
    out = _embed(x.astype(jnp.int32).reshape(-1), token_table, pos_table)
    return out.reshape(BATCH, MAXLEN, EMBED)
